# baseline (speedup 1.0000x reference)
import functools

import jax
import jax.numpy as jnp
from jax import lax
from jax.experimental import pallas as pl
from jax.experimental.pallas import tpu as pltpu
from jax.experimental.pallas import tpu_sc as plsc

NN = 10000
EE = 160000
NPAD = 10240
LANE = 128
NSUB = 16
ROWS_PER_TILE = NPAD // NSUB
RB = 1280
NRB = NPAD // RB

@functools.cache
def _mesh():
    return plsc.VectorSubcoreMesh(core_axis_name="c", subcore_axis_name="s",
                                  num_cores=2, num_subcores=NSUB)


DB = 40
DW = 128


def _deg_body(dst_hbm, ones_hbm, zeros_hbm, out_hbm,
              didx, didx_s, ones_v, ones_s, acc, sem):
    del sem
    cid = lax.axis_index("c")
    tid = lax.axis_index("s")
    e_half = EE // 2
    ept = e_half // NSUB
    nb = ept // 128
    rem = ept - nb * 128

    pltpu.sync_copy(ones_hbm, ones_v)
    pltpu.sync_copy(ones_hbm.at[pl.ds(0, rem)], ones_s)
    row0 = tid * ROWS_PER_TILE
    pltpu.sync_copy(zeros_hbm.at[pl.ds(0, ROWS_PER_TILE)],
                    acc.at[pl.ds(row0, ROWS_PER_TILE)])
    plsc.subcore_barrier()

    base = cid * e_half + tid * ept

    def step(i, carry):
        off = base + i * 128
        pltpu.sync_copy(dst_hbm.at[pl.ds(off, 128)], didx)
        pltpu.sync_copy(ones_v, acc.at[didx], add=True)
        return carry

    lax.fori_loop(0, nb, step, 0)
    off = base + nb * 128
    pltpu.sync_copy(dst_hbm.at[pl.ds(off, rem)], didx_s)
    pltpu.sync_copy(ones_s, acc.at[didx_s], add=True)

    plsc.subcore_barrier()
    pltpu.sync_copy(acc.at[pl.ds(row0, ROWS_PER_TILE)],
                    out_hbm.at[pl.ds(cid * NPAD + row0, ROWS_PER_TILE)])


@functools.cache
def _build_deg():
    return pl.kernel(
        _deg_body,
        out_type=jax.ShapeDtypeStruct((2 * NPAD, DW), jnp.float32),
        mesh=_mesh(),
        scratch_types=[
            pltpu.VMEM((128,), jnp.int32),
            pltpu.VMEM((8,), jnp.int32),
            pltpu.VMEM((128, DW), jnp.float32),
            pltpu.VMEM((8, DW), jnp.float32),
            pltpu.VMEM_SHARED((NPAD, DW), jnp.float32),
            pltpu.SemaphoreType.DMA,
        ],
    )


def _deg(dst, ones_rows, zeros_rows):
    return _build_deg()(dst, ones_rows, zeros_rows)


EB = 80


@functools.cache
def _make_agg(nchunks):
    ept = EE // NSUB
    nbatch = ept // EB
    npair = (nbatch - 1) // 2
    cpc = nchunks // 2

    nsteady = (nbatch - 5) // 4
    ntail = nbatch - 2 - 4 * nsteady

    def body(src_hbm, dst_hbm, zeros_hbm, hp, out, *rest):
        ss = rest[0:4]
        dd = rest[4:8]
        rr = rest[8:12]
        acc = rest[12]
        isem = rest[13:17]
        gsem = rest[17:21]
        ssem = rest[21:25]
        cid = lax.axis_index("c")
        tid = lax.axis_index("s")
        row0 = tid * ROWS_PER_TILE
        base = tid * ept

        def do_chunk(kofs):
            pltpu.sync_copy(zeros_hbm.at[pl.ds(0, ROWS_PER_TILE)],
                            acc.at[pl.ds(row0, ROWS_PER_TILE)])
            plsc.subcore_barrier()

            def idxload(i, b):
                off = base + i * EB
                pltpu.async_copy(src_hbm.at[pl.ds(off, EB)], ss[b], isem[b])
                pltpu.async_copy(dst_hbm.at[pl.ds(off, EB)], dd[b], isem[b])

            def gather(i, b):
                off = base + i * EB
                pltpu.make_async_copy(src_hbm.at[pl.ds(off, EB)], ss[b],
                                      isem[b]).wait()
                pltpu.make_async_copy(dst_hbm.at[pl.ds(off, EB)], dd[b],
                                      isem[b]).wait()
                if kofs:
                    for t in range(EB // 16):
                        sl = pl.ds(t * 16, 16)
                        ss[b][sl] = ss[b][sl] + kofs
                pltpu.async_copy(hp.at[ss[b]], rr[b], gsem[b])

            def scatter(b):
                pltpu.make_async_copy(hp.at[ss[b]], rr[b], gsem[b]).wait()
                pltpu.async_copy(rr[b], acc.at[dd[b]], ssem[b], add=True)

            def free(b):
                pltpu.make_async_copy(rr[b], acc.at[dd[b]], ssem[b]).wait()

            idxload(0, 0)
            idxload(1, 1)
            gather(0, 0)
            idxload(2, 2)
            gather(1, 1)
            scatter(0)
            idxload(3, 3)

            def steady(j, c):
                i0 = 2 + 4 * j
                for r in range(4):
                    i = i0 + r
                    b = (2 + r) % 4
                    gather(i, b)
                    scatter((b - 1) % 4)
                    free((b - 2) % 4)
                    idxload(i + 2, (b + 2) % 4)
                return c

            lax.fori_loop(0, nsteady, steady, 0)

            i0 = 2 + 4 * nsteady
            for r in range(ntail):
                i = i0 + r
                b = (2 + r) % 4
                gather(i, b)
                scatter((b - 1) % 4)
                free((b - 2) % 4)
                if i + 2 < nbatch:
                    idxload(i + 2, (b + 2) % 4)
            blast = (2 + ntail - 1) % 4
            scatter(blast)
            free((blast - 1) % 4)
            free(blast)

            plsc.subcore_barrier()
            pltpu.sync_copy(acc.at[pl.ds(row0, ROWS_PER_TILE)],
                            out.at[pl.ds(kofs + row0, ROWS_PER_TILE)])

        for k in range(nchunks):
            @pl.when(cid == (k // cpc))
            def _(k=k):
                do_chunk(k * NPAD)

    return pl.kernel(
        body,
        out_type=jax.ShapeDtypeStruct((nchunks * NPAD, LANE), jnp.float32),
        mesh=_mesh(),
        scratch_types=(
            [pltpu.VMEM((EB,), jnp.int32) for _ in range(8)]
            + [pltpu.VMEM((EB, LANE), jnp.float32) for _ in range(4)]
            + [pltpu.VMEM_SHARED((NPAD, LANE), jnp.float32)]
            + [pltpu.SemaphoreType.DMA for _ in range(12)]
        ),
    )


def _agg4(src, dst, zeros_rows, hp_flat):
    return _make_agg(4)(src, dst, zeros_rows, hp_flat)


def _agg2(src, dst, zeros_rows, hp_flat):
    return _make_agg(2)(src, dst, zeros_rows, hp_flat)


def _dinv_of(dp):
    return lax.rsqrt(dp[0] + dp[1] + 1.0)


def _k1_body(x_ref, w_ref, dp_ref, out_ref):
    dinv = _dinv_of(dp_ref[...])
    p = jnp.dot(x_ref[...], w_ref[...], preferred_element_type=jnp.float32)
    out_ref[0] = p * dinv


def _k1(xp, w1, dp):
    return pl.pallas_call(
        _k1_body,
        grid=(NRB, 4),
        in_specs=[
            pl.BlockSpec((RB, 256), lambda i, c: (i, 0)),
            pl.BlockSpec((256, LANE), lambda i, c: (0, c)),
            pl.BlockSpec((2, RB, DW), lambda i, c: (0, i, 0)),
        ],
        out_specs=pl.BlockSpec((1, RB, LANE), lambda i, c: (c, i, 0)),
        out_shape=jax.ShapeDtypeStruct((4, NPAD, LANE), jnp.float32),
    )(xp, w1, dp)


def _k3_body(agg_ref, hp_ref, dp_ref, b1_ref, w2_ref, out_ref):
    dinv = _dinv_of(dp_ref[...])
    acc = jnp.zeros((RB, LANE), jnp.float32)
    for kc in range(4):
        t = jnp.tanh((agg_ref[kc] + hp_ref[kc]) * dinv + b1_ref[kc][None, :])
        acc = acc + jnp.dot(t, w2_ref[kc], preferred_element_type=jnp.float32)
    out_ref[0] = acc * dinv


def _k3(agg1, hp1, dp, b1, w2):
    return pl.pallas_call(
        _k3_body,
        grid=(NRB, 2),
        in_specs=[
            pl.BlockSpec((4, RB, LANE), lambda i, c: (0, i, 0)),
            pl.BlockSpec((4, RB, LANE), lambda i, c: (0, i, 0)),
            pl.BlockSpec((2, RB, DW), lambda i, c: (0, i, 0)),
            pl.BlockSpec((4, LANE), lambda i, c: (0, 0)),
            pl.BlockSpec((4, 128, LANE), lambda i, c: (0, 0, c)),
        ],
        out_specs=pl.BlockSpec((1, RB, LANE), lambda i, c: (c, i, 0)),
        out_shape=jax.ShapeDtypeStruct((2, NPAD, LANE), jnp.float32),
    )(agg1, hp1, dp, b1, w2)


def _k5_body(agg_ref, hp_ref, dp_ref, b2_ref,
             wc1_ref, bc1_ref, gc1_ref, bec1_ref,
             wc2_ref, bc2_ref, gc2_ref, bec2_ref,
             wc3_ref, bc3_ref,
             wr1_ref, br1_ref, wr2_ref, br2_ref, wr3_ref, br3_ref,
             c_ref, r_ref, h_ref):
    dinv = _dinv_of(dp_ref[...])
    h0 = jnp.tanh((agg_ref[0] + hp_ref[0]) * dinv + b2_ref[0][None, :])
    h1 = jnp.tanh((agg_ref[1] + hp_ref[1]) * dinv + b2_ref[1][None, :])
    h = jnp.concatenate([h0, h1], axis=1)
    h_ref[...] = h
    bn = 1.0 / jnp.sqrt(1.0 + 1e-5)
    c1 = jnp.dot(h, wc1_ref[...], preferred_element_type=jnp.float32) + bc1_ref[...]
    c1 = jnp.maximum(c1 * bn * gc1_ref[...] + bec1_ref[...], 0.0)
    c2 = jnp.dot(c1, wc2_ref[...], preferred_element_type=jnp.float32) + bc2_ref[...]
    c2 = jnp.maximum(c2 * bn * gc2_ref[...] + bec2_ref[...], 0.0)
    lg = jnp.dot(c2, wc3_ref[...], preferred_element_type=jnp.float32) + bc3_ref[...]
    lg = lg - jnp.max(lg, axis=1, keepdims=True)
    e = jnp.exp(lg)
    c_ref[...] = e / jnp.sum(e, axis=1, keepdims=True)
    r1 = jnp.maximum(
        jnp.dot(h, wr1_ref[...], preferred_element_type=jnp.float32) + br1_ref[...], 0.0)
    r2 = jnp.maximum(
        jnp.dot(r1, wr2_ref[...], preferred_element_type=jnp.float32) + br2_ref[...], 0.0)
    r_ref[...] = jnp.dot(r2, wr3_ref[...], preferred_element_type=jnp.float32) + br3_ref[...]


def _k5(agg2, hp2, dp, b2, wc1, bc1, gc1, bec1, wc2, bc2, gc2, bec2, wc3, bc3,
        wr1, br1, wr2, br2, wr3, br3):
    full = lambda shape: pl.BlockSpec(shape, lambda i: tuple(0 for _ in shape))
    rbo = 2000
    return pl.pallas_call(
        _k5_body,
        grid=(NN // rbo,),
        in_specs=[
            pl.BlockSpec((2, rbo, LANE), lambda i: (0, i, 0)),
            pl.BlockSpec((2, rbo, LANE), lambda i: (0, i, 0)),
            pl.BlockSpec((2, rbo, DW), lambda i: (0, i, 0)),
            full((2, LANE)),
            full((256, 256)), full((1, 256)), full((1, 256)), full((1, 256)),
            full((256, 128)), full((1, 128)), full((1, 128)), full((1, 128)),
            full((128, 16)), full((1, 16)),
            full((256, 256)), full((1, 256)),
            full((256, 512)), full((1, 512)),
            full((512, 256)), full((1, 256)),
        ],
        out_specs=[
            pl.BlockSpec((rbo, 16), lambda i: (i, 0)),
            pl.BlockSpec((rbo, 256), lambda i: (i, 0)),
            pl.BlockSpec((rbo, 256), lambda i: (i, 0)),
        ],
        out_shape=[
            jax.ShapeDtypeStruct((NN, 16), jnp.float32),
            jax.ShapeDtypeStruct((NN, 256), jnp.float32),
            jax.ShapeDtypeStruct((NN, 256), jnp.float32),
        ],
    )(agg2, hp2, dp, b2, wc1, bc1, gc1, bec1, wc2, bc2, gc2, bec2, wc3, bc3,
      wr1, br1, wr2, br2, wr3, br3)


def kernel(x, edge_index, W_gc1, b_gc1, W_gc2, b_gc2, W_c1, b_c1, g_c1, be_c1,
           W_c2, b_c2, g_c2, be_c2, W_c3, b_c3, W_r1, b_r1, W_r2, b_r2,
           W_r3, b_r3):
    src = edge_index[0]
    dst = edge_index[1]
    zeros_rows = jnp.zeros((ROWS_PER_TILE, LANE), jnp.float32)
    zeros_deg = jnp.zeros((ROWS_PER_TILE, DW), jnp.float32)
    ones_deg = jnp.ones((128, DW), jnp.float32)

    dp = _deg(dst, ones_deg, zeros_deg).reshape(2, NPAD, DW)

    hp1 = _k1(x, W_gc1, dp)
    agg1 = _agg4(src, dst, zeros_rows,
                 hp1.reshape(4 * NPAD, LANE)).reshape(4, NPAD, LANE)

    hp2 = _k3(agg1, hp1, dp, b_gc1.reshape(4, LANE),
              W_gc2.reshape(4, 128, 256))
    agg2 = _agg2(src, dst, zeros_rows,
                 hp2.reshape(2 * NPAD, LANE)).reshape(2, NPAD, LANE)

    return _k5(
        agg2, hp2, dp, b_gc2.reshape(2, LANE),
        W_c1, b_c1.reshape(1, 256), g_c1.reshape(1, 256), be_c1.reshape(1, 256),
        W_c2, b_c2.reshape(1, 128), g_c2.reshape(1, 128), be_c2.reshape(1, 128),
        W_c3, b_c3.reshape(1, 16),
        W_r1, b_r1.reshape(1, 256),
        W_r2, b_r2.reshape(1, 512),
        W_r3, b_r3.reshape(1, 256))


# speedup vs baseline: 1.0391x; 1.0391x over previous
import functools

import jax
import jax.numpy as jnp
from jax import lax
from jax.experimental import pallas as pl
from jax.experimental.pallas import tpu as pltpu
from jax.experimental.pallas import tpu_sc as plsc

NN = 10000
EE = 160000
NPAD = 10240
LANE = 128
NSUB = 16
ROWS_PER_TILE = NPAD // NSUB
RB = 1280
NRB = NPAD // RB

@functools.cache
def _mesh():
    return plsc.VectorSubcoreMesh(core_axis_name="c", subcore_axis_name="s",
                                  num_cores=2, num_subcores=NSUB)


DB = 40
DW = 128


def _deg_body(dst_hbm, ones_hbm, zeros_hbm, out_hbm, *rest):
    dd = rest[0:4]
    ones_v = rest[4]
    acc = rest[5]
    isem = rest[6:10]
    ssem = rest[10:14]
    cid = lax.axis_index("c")
    tid = lax.axis_index("s")
    e_half = EE // 2
    ept = e_half // NSUB
    nbatch = ept // DB
    nsteady = (nbatch - 5) // 4
    ntail = nbatch - 2 - 4 * nsteady

    row0 = tid * ROWS_PER_TILE
    base = cid * e_half + tid * ept
    pltpu.sync_copy(ones_hbm, ones_v)
    pltpu.sync_copy(zeros_hbm.at[pl.ds(0, ROWS_PER_TILE)],
                    acc.at[pl.ds(row0, ROWS_PER_TILE)])
    plsc.subcore_barrier()

    def idxload(i, b):
        pltpu.async_copy(dst_hbm.at[pl.ds(base + i * DB, DB)], dd[b], isem[b])

    def scat(i, b):
        pltpu.make_async_copy(dst_hbm.at[pl.ds(base + i * DB, DB)], dd[b],
                              isem[b]).wait()
        pltpu.async_copy(ones_v, acc.at[dd[b]], ssem[b], add=True)

    def free(b):
        pltpu.make_async_copy(ones_v, acc.at[dd[b]], ssem[b]).wait()

    idxload(0, 0)
    idxload(1, 1)
    scat(0, 0)
    idxload(2, 2)
    scat(1, 1)
    idxload(3, 3)

    def steady(j, c):
        i0 = 2 + 4 * j
        for r in range(4):
            i = i0 + r
            b = (2 + r) % 4
            scat(i, b)
            free((b - 2) % 4)
            idxload(i + 2, (b + 2) % 4)
        return c

    lax.fori_loop(0, nsteady, steady, 0)
    i0 = 2 + 4 * nsteady
    for r in range(ntail):
        i = i0 + r
        b = (2 + r) % 4
        scat(i, b)
        free((b - 2) % 4)
        if i + 2 < nbatch:
            idxload(i + 2, (b + 2) % 4)
    blast = (2 + ntail - 1) % 4
    free((blast - 1) % 4)
    free(blast)

    plsc.subcore_barrier()
    pltpu.sync_copy(acc.at[pl.ds(row0, ROWS_PER_TILE)],
                    out_hbm.at[pl.ds(cid * NPAD + row0, ROWS_PER_TILE)])


@functools.cache
def _build_deg():
    return pl.kernel(
        _deg_body,
        out_type=jax.ShapeDtypeStruct((2 * NPAD, DW), jnp.float32),
        mesh=_mesh(),
        scratch_types=(
            [pltpu.VMEM((DB,), jnp.int32) for _ in range(4)]
            + [pltpu.VMEM((DB, DW), jnp.float32)]
            + [pltpu.VMEM_SHARED((NPAD, DW), jnp.float32)]
            + [pltpu.SemaphoreType.DMA for _ in range(8)]
        ),
    )


def _deg(dst, ones_rows, zeros_rows):
    return _build_deg()(dst, ones_rows, zeros_rows)


EB = 80


@functools.cache
def _make_agg(nchunks):
    ept = EE // NSUB
    nbatch = ept // EB
    npair = (nbatch - 1) // 2
    cpc = nchunks // 2

    nsteady = (nbatch - 5) // 4
    ntail = nbatch - 2 - 4 * nsteady

    def body(src_hbm, dst_hbm, zeros_hbm, hp, out, *rest):
        ss = rest[0:4]
        dd = rest[4:8]
        rr = rest[8:12]
        acc = rest[12]
        isem = rest[13:17]
        gsem = rest[17:21]
        ssem = rest[21:25]
        cid = lax.axis_index("c")
        tid = lax.axis_index("s")
        row0 = tid * ROWS_PER_TILE
        base = tid * ept

        def do_chunk(kofs):
            pltpu.sync_copy(zeros_hbm.at[pl.ds(0, ROWS_PER_TILE)],
                            acc.at[pl.ds(row0, ROWS_PER_TILE)])
            plsc.subcore_barrier()

            def idxload(i, b):
                off = base + i * EB
                pltpu.async_copy(src_hbm.at[pl.ds(off, EB)], ss[b], isem[b])
                pltpu.async_copy(dst_hbm.at[pl.ds(off, EB)], dd[b], isem[b])

            def gather(i, b):
                off = base + i * EB
                pltpu.make_async_copy(src_hbm.at[pl.ds(off, EB)], ss[b],
                                      isem[b]).wait()
                pltpu.make_async_copy(dst_hbm.at[pl.ds(off, EB)], dd[b],
                                      isem[b]).wait()
                if kofs:
                    for t in range(EB // 16):
                        sl = pl.ds(t * 16, 16)
                        ss[b][sl] = ss[b][sl] + kofs
                pltpu.async_copy(hp.at[ss[b]], rr[b], gsem[b])

            def scatter(b):
                pltpu.make_async_copy(hp.at[ss[b]], rr[b], gsem[b]).wait()
                pltpu.async_copy(rr[b], acc.at[dd[b]], ssem[b], add=True)

            def free(b):
                pltpu.make_async_copy(rr[b], acc.at[dd[b]], ssem[b]).wait()

            idxload(0, 0)
            idxload(1, 1)
            gather(0, 0)
            idxload(2, 2)
            gather(1, 1)
            scatter(0)
            idxload(3, 3)

            def steady(j, c):
                i0 = 2 + 4 * j
                for r in range(4):
                    i = i0 + r
                    b = (2 + r) % 4
                    gather(i, b)
                    scatter((b - 1) % 4)
                    free((b - 2) % 4)
                    idxload(i + 2, (b + 2) % 4)
                return c

            lax.fori_loop(0, nsteady, steady, 0)

            i0 = 2 + 4 * nsteady
            for r in range(ntail):
                i = i0 + r
                b = (2 + r) % 4
                gather(i, b)
                scatter((b - 1) % 4)
                free((b - 2) % 4)
                if i + 2 < nbatch:
                    idxload(i + 2, (b + 2) % 4)
            blast = (2 + ntail - 1) % 4
            scatter(blast)
            free((blast - 1) % 4)
            free(blast)

            plsc.subcore_barrier()
            pltpu.sync_copy(acc.at[pl.ds(row0, ROWS_PER_TILE)],
                            out.at[pl.ds(kofs + row0, ROWS_PER_TILE)])

        for k in range(nchunks):
            @pl.when(cid == (k // cpc))
            def _(k=k):
                do_chunk(k * NPAD)

    return pl.kernel(
        body,
        out_type=jax.ShapeDtypeStruct((nchunks * NPAD, LANE), jnp.float32),
        mesh=_mesh(),
        scratch_types=(
            [pltpu.VMEM((EB,), jnp.int32) for _ in range(8)]
            + [pltpu.VMEM((EB, LANE), jnp.float32) for _ in range(4)]
            + [pltpu.VMEM_SHARED((NPAD, LANE), jnp.float32)]
            + [pltpu.SemaphoreType.DMA for _ in range(12)]
        ),
    )


def _agg4(src, dst, zeros_rows, hp_flat):
    return _make_agg(4)(src, dst, zeros_rows, hp_flat)


def _agg2(src, dst, zeros_rows, hp_flat):
    return _make_agg(2)(src, dst, zeros_rows, hp_flat)


def _dinv_of(dp):
    return lax.rsqrt(dp[0] + dp[1] + 1.0)


def _k1_body(x_ref, w_ref, dp_ref, out_ref):
    dinv = _dinv_of(dp_ref[...])
    p = jnp.dot(x_ref[...], w_ref[...], preferred_element_type=jnp.float32)
    out_ref[0] = p * dinv


def _k1(xp, w1, dp):
    return pl.pallas_call(
        _k1_body,
        grid=(NRB, 4),
        in_specs=[
            pl.BlockSpec((RB, 256), lambda i, c: (i, 0)),
            pl.BlockSpec((256, LANE), lambda i, c: (0, c)),
            pl.BlockSpec((2, RB, DW), lambda i, c: (0, i, 0)),
        ],
        out_specs=pl.BlockSpec((1, RB, LANE), lambda i, c: (c, i, 0)),
        out_shape=jax.ShapeDtypeStruct((4, NPAD, LANE), jnp.float32),
    )(xp, w1, dp)


def _k3_body(agg_ref, hp_ref, dp_ref, b1_ref, w2_ref, out_ref):
    dinv = _dinv_of(dp_ref[...])
    acc = jnp.zeros((RB, LANE), jnp.float32)
    for kc in range(4):
        t = jnp.tanh((agg_ref[kc] + hp_ref[kc]) * dinv + b1_ref[kc][None, :])
        acc = acc + jnp.dot(t, w2_ref[kc], preferred_element_type=jnp.float32)
    out_ref[0] = acc * dinv


def _k3(agg1, hp1, dp, b1, w2):
    return pl.pallas_call(
        _k3_body,
        grid=(NRB, 2),
        in_specs=[
            pl.BlockSpec((4, RB, LANE), lambda i, c: (0, i, 0)),
            pl.BlockSpec((4, RB, LANE), lambda i, c: (0, i, 0)),
            pl.BlockSpec((2, RB, DW), lambda i, c: (0, i, 0)),
            pl.BlockSpec((4, LANE), lambda i, c: (0, 0)),
            pl.BlockSpec((4, 128, LANE), lambda i, c: (0, 0, c)),
        ],
        out_specs=pl.BlockSpec((1, RB, LANE), lambda i, c: (c, i, 0)),
        out_shape=jax.ShapeDtypeStruct((2, NPAD, LANE), jnp.float32),
    )(agg1, hp1, dp, b1, w2)


def _k5_body(agg_ref, hp_ref, dp_ref, b2_ref,
             wc1_ref, bc1_ref, gc1_ref, bec1_ref,
             wc2_ref, bc2_ref, gc2_ref, bec2_ref,
             wc3_ref, bc3_ref,
             wr1_ref, br1_ref, wr2_ref, br2_ref, wr3_ref, br3_ref,
             c_ref, r_ref, h_ref):
    dinv = _dinv_of(dp_ref[...])
    h0 = jnp.tanh((agg_ref[0] + hp_ref[0]) * dinv + b2_ref[0][None, :])
    h1 = jnp.tanh((agg_ref[1] + hp_ref[1]) * dinv + b2_ref[1][None, :])
    h = jnp.concatenate([h0, h1], axis=1)
    h_ref[...] = h
    bn = 1.0 / jnp.sqrt(1.0 + 1e-5)
    c1 = jnp.dot(h, wc1_ref[...], preferred_element_type=jnp.float32) + bc1_ref[...]
    c1 = jnp.maximum(c1 * bn * gc1_ref[...] + bec1_ref[...], 0.0)
    c2 = jnp.dot(c1, wc2_ref[...], preferred_element_type=jnp.float32) + bc2_ref[...]
    c2 = jnp.maximum(c2 * bn * gc2_ref[...] + bec2_ref[...], 0.0)
    lg = jnp.dot(c2, wc3_ref[...], preferred_element_type=jnp.float32) + bc3_ref[...]
    lg = lg - jnp.max(lg, axis=1, keepdims=True)
    e = jnp.exp(lg)
    c_ref[...] = e / jnp.sum(e, axis=1, keepdims=True)
    r1 = jnp.maximum(
        jnp.dot(h, wr1_ref[...], preferred_element_type=jnp.float32) + br1_ref[...], 0.0)
    r2 = jnp.maximum(
        jnp.dot(r1, wr2_ref[...], preferred_element_type=jnp.float32) + br2_ref[...], 0.0)
    r_ref[...] = jnp.dot(r2, wr3_ref[...], preferred_element_type=jnp.float32) + br3_ref[...]


def _k5(agg2, hp2, dp, b2, wc1, bc1, gc1, bec1, wc2, bc2, gc2, bec2, wc3, bc3,
        wr1, br1, wr2, br2, wr3, br3):
    full = lambda shape: pl.BlockSpec(shape, lambda i: tuple(0 for _ in shape))
    rbo = 2000
    return pl.pallas_call(
        _k5_body,
        grid=(NN // rbo,),
        in_specs=[
            pl.BlockSpec((2, rbo, LANE), lambda i: (0, i, 0)),
            pl.BlockSpec((2, rbo, LANE), lambda i: (0, i, 0)),
            pl.BlockSpec((2, rbo, DW), lambda i: (0, i, 0)),
            full((2, LANE)),
            full((256, 256)), full((1, 256)), full((1, 256)), full((1, 256)),
            full((256, 128)), full((1, 128)), full((1, 128)), full((1, 128)),
            full((128, 16)), full((1, 16)),
            full((256, 256)), full((1, 256)),
            full((256, 512)), full((1, 512)),
            full((512, 256)), full((1, 256)),
        ],
        out_specs=[
            pl.BlockSpec((rbo, 16), lambda i: (i, 0)),
            pl.BlockSpec((rbo, 256), lambda i: (i, 0)),
            pl.BlockSpec((rbo, 256), lambda i: (i, 0)),
        ],
        out_shape=[
            jax.ShapeDtypeStruct((NN, 16), jnp.float32),
            jax.ShapeDtypeStruct((NN, 256), jnp.float32),
            jax.ShapeDtypeStruct((NN, 256), jnp.float32),
        ],
    )(agg2, hp2, dp, b2, wc1, bc1, gc1, bec1, wc2, bc2, gc2, bec2, wc3, bc3,
      wr1, br1, wr2, br2, wr3, br3)


def kernel(x, edge_index, W_gc1, b_gc1, W_gc2, b_gc2, W_c1, b_c1, g_c1, be_c1,
           W_c2, b_c2, g_c2, be_c2, W_c3, b_c3, W_r1, b_r1, W_r2, b_r2,
           W_r3, b_r3):
    src = edge_index[0]
    dst = edge_index[1]
    zeros_rows = jnp.zeros((ROWS_PER_TILE, LANE), jnp.float32)
    zeros_deg = jnp.zeros((ROWS_PER_TILE, DW), jnp.float32)
    ones_deg = jnp.ones((DB, DW), jnp.float32)

    dp = _deg(dst, ones_deg, zeros_deg).reshape(2, NPAD, DW)

    hp1 = _k1(x, W_gc1, dp)
    agg1 = _agg4(src, dst, zeros_rows,
                 hp1.reshape(4 * NPAD, LANE)).reshape(4, NPAD, LANE)

    hp2 = _k3(agg1, hp1, dp, b_gc1.reshape(4, LANE),
              W_gc2.reshape(4, 128, 256))
    agg2 = _agg2(src, dst, zeros_rows,
                 hp2.reshape(2 * NPAD, LANE)).reshape(2, NPAD, LANE)

    return _k5(
        agg2, hp2, dp, b_gc2.reshape(2, LANE),
        W_c1, b_c1.reshape(1, 256), g_c1.reshape(1, 256), be_c1.reshape(1, 256),
        W_c2, b_c2.reshape(1, 128), g_c2.reshape(1, 128), be_c2.reshape(1, 128),
        W_c3, b_c3.reshape(1, 16),
        W_r1, b_r1.reshape(1, 256),
        W_r2, b_r2.reshape(1, 512),
        W_r3, b_r3.reshape(1, 256))
